# baseline (speedup 1.0000x reference)
import functools

import jax
import jax.numpy as jnp
from jax.experimental import pallas as pl
from jax.experimental.pallas import tpu as pltpu

_TOPK = 0.1
_LAMBDA_LOCALITY = 0.5
_LAMBDA_ACT_L1 = 1.0

_INTERPRET = False


def _sobel_mag(a):
    B, H, W = a.shape
    p = jnp.pad(a, ((0, 0), (1, 1), (1, 1)))
    D = p[:, :, :-2] - p[:, :, 2:]
    S = p[:, :, :-2] + 2.0 * p[:, :, 1:-1] + p[:, :, 2:]
    gx = D[:, 0:H] + 2.0 * D[:, 1:H + 1] + D[:, 2:H + 2]
    gy = S[:, 0:H] - S[:, 2:H + 2]
    m2 = gx * gx + gy * gy
    return jnp.sqrt(m2), m2


def _plane_kernel(k_top, x_ref, out_ref, rs_ref, cs_ref):
    a = x_ref[...]
    B = a.shape[0]
    ab = a.astype(jnp.bfloat16).astype(jnp.float32)
    mag, m2 = _sobel_mag(ab)
    rs_ref[...] = jnp.sum(mag, axis=2)
    cs_ref[...] = jnp.sum(mag, axis=1)

    bits = jax.lax.bitcast_convert_type(mag, jnp.int32)
    npix = mag.shape[1] * mag.shape[2]

    def probe(t, lo, hi):
        cnt = jnp.sum((bits >= t).astype(jnp.int32), axis=(1, 2),
                      keepdims=True)
        ge = cnt >= k_top
        return jnp.where(ge, t, lo), jnp.where(ge, hi, t)

    sumsq = jnp.sum(m2, axis=(1, 2), keepdims=True)
    t_hat = jnp.sqrt(sumsq * (-jnp.log(jnp.float32(_TOPK)) / npix))
    t_hi = jax.lax.bitcast_convert_type(t_hat * 1.03, jnp.int32)
    t_lo = jax.lax.bitcast_convert_type(t_hat * 0.97, jnp.int32)

    lo = jnp.zeros((B, 1, 1), jnp.int32)
    hi = jnp.full((B, 1, 1), 0x7F800001, jnp.int32)
    lo, hi = probe(jnp.clip(t_hi, lo + 1, hi - 1), lo, hi)
    lo, hi = probe(jnp.clip(t_lo, lo + 1, hi - 1), lo, hi)

    def condA(carry):
        lo, hi = carry
        return jnp.any(hi - lo > 65535)

    def bodyA(carry):
        lo, hi = carry
        return probe(lo + (hi - lo) // 2, lo, hi)

    lo, hi = jax.lax.while_loop(condA, bodyA, (lo, hi))

    base = lo
    r16 = (jnp.clip(bits - base, 0, 65535) - 32768).astype(jnp.int16)

    def probe16(t, lo, hi):
        tb = (t - base - 32768).astype(jnp.int16)
        rows = jnp.sum((r16 >= tb).astype(jnp.int16), axis=2)
        cnt = jnp.sum(rows.astype(jnp.int32), axis=1)[:, None, None]
        ge = cnt >= k_top
        return jnp.where(ge, t, lo), jnp.where(ge, hi, t)

    def condB(carry):
        lo, hi = carry
        return jnp.any(hi - lo > 1)

    def bodyB(carry):
        lo, hi = carry
        return probe16(lo + (hi - lo) // 2, lo, hi)

    lo, hi = jax.lax.while_loop(condB, bodyB, (lo, hi))
    out_ref[...] = jnp.where(bits >= lo, a, 0.0)


def _reg_kernel(total_elems, rs_ref, cs_ref, out_ref):
    rs = rs_ref[...]
    cs = cs_ref[...]
    s = jnp.sum(rs, axis=2)

    def ent(prob):
        p = prob / s[:, :, None]
        logp = jnp.log(jnp.clip(p, 1e-38, None))
        return -jnp.sum(p * logp, axis=2)

    ex = ent(rs)
    ey = ent(cs)
    tot = jnp.sum(s, axis=1, keepdims=True)
    w = s / tot
    reg = (jnp.sum(s) / total_elems * _LAMBDA_ACT_L1
           + (jnp.mean(ex * w) + jnp.mean(ey * w)) * _LAMBDA_LOCALITY)
    out_ref[...] = reg.reshape(1, 1)


def kernel(x, tau):
    n, c, h, w = x.shape
    nc = n * c
    k_top = max(int(_TOPK * h * w), 1)
    xr = x.reshape(nc, h, w)

    B = 8
    assert nc % B == 0
    grid = (nc // B,)
    sparse, rs, cs = pl.pallas_call(
        functools.partial(_plane_kernel, k_top),
        grid=grid,
        in_specs=[pl.BlockSpec((B, h, w), lambda i: (i, 0, 0))],
        out_specs=[
            pl.BlockSpec((B, h, w), lambda i: (i, 0, 0)),
            pl.BlockSpec((B, h), lambda i: (i, 0)),
            pl.BlockSpec((B, w), lambda i: (i, 0)),
        ],
        out_shape=[
            jax.ShapeDtypeStruct((nc, h, w), x.dtype),
            jax.ShapeDtypeStruct((nc, h), jnp.float32),
            jax.ShapeDtypeStruct((nc, w), jnp.float32),
        ],
        interpret=_INTERPRET,
    )(xr)

    reg2d = pl.pallas_call(
        functools.partial(_reg_kernel, float(nc * h * w)),
        out_shape=jax.ShapeDtypeStruct((1, 1), jnp.float32),
        interpret=_INTERPRET,
    )(rs.reshape(n, c, h), cs.reshape(n, c, w))

    return sparse.reshape(n, c, h, w), reg2d[0, 0]


# speedup vs baseline: 2.1800x; 2.1800x over previous
import functools

import jax
import jax.numpy as jnp
from jax.experimental import pallas as pl
from jax.experimental.pallas import tpu as pltpu

_TOPK = 0.1
_LAMBDA_LOCALITY = 0.5
_LAMBDA_ACT_L1 = 1.0

_INTERPRET = False


def _sobel_mag(a):
    B, H, W = a.shape
    p = jnp.pad(a, ((0, 0), (1, 1), (1, 1)))
    D = p[:, :, :-2] - p[:, :, 2:]
    S = p[:, :, :-2] + 2.0 * p[:, :, 1:-1] + p[:, :, 2:]
    gx = D[:, 0:H] + 2.0 * D[:, 1:H + 1] + D[:, 2:H + 2]
    gy = S[:, 0:H] - S[:, 2:H + 2]
    m2 = gx * gx + gy * gy
    return jnp.sqrt(m2), m2


def _plane_kernel(k_top, x_ref, out_ref, rs_ref, cs_ref):
    a = x_ref[...]
    B = a.shape[0]
    ab = a.astype(jnp.bfloat16).astype(jnp.float32)
    mag, m2 = _sobel_mag(ab)
    rs_ref[...] = jnp.sum(mag, axis=2)
    cs_ref[...] = jnp.sum(mag, axis=1)

    bits = jax.lax.bitcast_convert_type(mag, jnp.int32)
    npix = mag.shape[1] * mag.shape[2]

    def probe(t, lo, hi):
        cnt = jnp.sum((bits >= t).astype(jnp.int32), axis=(1, 2),
                      keepdims=True)
        ge = cnt >= k_top
        return jnp.where(ge, t, lo), jnp.where(ge, hi, t)

    sumsq = jnp.sum(m2, axis=(1, 2), keepdims=True)
    t_hat = jnp.sqrt(sumsq * (-jnp.log(jnp.float32(_TOPK)) / npix))
    t_hi = jax.lax.bitcast_convert_type(t_hat * 1.03, jnp.int32)
    t_lo = jax.lax.bitcast_convert_type(t_hat * 0.97, jnp.int32)

    lo = jnp.zeros((B, 1, 1), jnp.int32)
    hi = jnp.full((B, 1, 1), 0x7F800001, jnp.int32)
    lo, hi = probe(jnp.clip(t_hi, lo + 1, hi - 1), lo, hi)
    lo, hi = probe(jnp.clip(t_lo, lo + 1, hi - 1), lo, hi)

    def cond(carry):
        lo, hi = carry
        return jnp.any(hi - lo > 1)

    def body(carry):
        lo, hi = carry
        return probe(lo + (hi - lo) // 2, lo, hi)

    lo, hi = jax.lax.while_loop(cond, body, (lo, hi))
    out_ref[...] = jnp.where(bits >= lo, a, 0.0)


def _reg_kernel(total_elems, rs_ref, cs_ref, out_ref):
    rs = rs_ref[...]
    cs = cs_ref[...]
    s = jnp.sum(rs, axis=2)

    def ent(prob):
        p = prob / s[:, :, None]
        logp = jnp.log(jnp.clip(p, 1e-38, None))
        return -jnp.sum(p * logp, axis=2)

    ex = ent(rs)
    ey = ent(cs)
    tot = jnp.sum(s, axis=1, keepdims=True)
    w = s / tot
    reg = (jnp.sum(s) / total_elems * _LAMBDA_ACT_L1
           + (jnp.mean(ex * w) + jnp.mean(ey * w)) * _LAMBDA_LOCALITY)
    out_ref[...] = reg.reshape(1, 1)


def kernel(x, tau):
    n, c, h, w = x.shape
    nc = n * c
    k_top = max(int(_TOPK * h * w), 1)
    xr = x.reshape(nc, h, w)

    B = 16
    assert nc % B == 0
    grid = (nc // B,)
    sparse, rs, cs = pl.pallas_call(
        functools.partial(_plane_kernel, k_top),
        grid=grid,
        in_specs=[pl.BlockSpec((B, h, w), lambda i: (i, 0, 0))],
        out_specs=[
            pl.BlockSpec((B, h, w), lambda i: (i, 0, 0)),
            pl.BlockSpec((B, h), lambda i: (i, 0)),
            pl.BlockSpec((B, w), lambda i: (i, 0)),
        ],
        out_shape=[
            jax.ShapeDtypeStruct((nc, h, w), x.dtype),
            jax.ShapeDtypeStruct((nc, h), jnp.float32),
            jax.ShapeDtypeStruct((nc, w), jnp.float32),
        ],
        interpret=_INTERPRET,
    )(xr)

    reg2d = pl.pallas_call(
        functools.partial(_reg_kernel, float(nc * h * w)),
        out_shape=jax.ShapeDtypeStruct((1, 1), jnp.float32),
        interpret=_INTERPRET,
    )(rs.reshape(n, c, h), cs.reshape(n, c, w))

    return sparse.reshape(n, c, h, w), reg2d[0, 0]


# speedup vs baseline: 2.3258x; 1.0669x over previous
import functools

import jax
import jax.numpy as jnp
from jax.experimental import pallas as pl
from jax.experimental.pallas import tpu as pltpu

_TOPK = 0.1
_LAMBDA_LOCALITY = 0.5
_LAMBDA_ACT_L1 = 1.0

_INTERPRET = False


def _sobel_mag(a):
    B, H, W = a.shape
    p = jnp.pad(a, ((0, 0), (1, 1), (1, 1)))
    D = p[:, :, :-2] - p[:, :, 2:]
    S = p[:, :, :-2] + 2.0 * p[:, :, 1:-1] + p[:, :, 2:]
    gx = D[:, 0:H] + 2.0 * D[:, 1:H + 1] + D[:, 2:H + 2]
    gy = S[:, 0:H] - S[:, 2:H + 2]
    m2 = gx * gx + gy * gy
    return jnp.sqrt(m2), m2


def _plane_kernel(k_top, x_ref, out_ref, rs_ref, cs_ref):
    a = x_ref[...]
    B = a.shape[0]
    ab = a.astype(jnp.bfloat16).astype(jnp.float32)
    mag, m2 = _sobel_mag(ab)
    rs_ref[...] = jnp.sum(mag, axis=2)
    cs_ref[...] = jnp.sum(mag, axis=1)

    bits = jax.lax.bitcast_convert_type(mag, jnp.int32)
    npix = mag.shape[1] * mag.shape[2]

    def probe(t, lo, hi):
        cnt = jnp.sum((bits >= t).astype(jnp.int32), axis=(1, 2),
                      keepdims=True)
        ge = cnt >= k_top
        return jnp.where(ge, t, lo), jnp.where(ge, hi, t)

    sumsq = jnp.sum(m2, axis=(1, 2), keepdims=True)
    t_hat = jnp.sqrt(sumsq * (-jnp.log(jnp.float32(_TOPK)) / npix))
    t_hi = jax.lax.bitcast_convert_type(t_hat * 1.03, jnp.int32)
    t_lo = jax.lax.bitcast_convert_type(t_hat * 0.97, jnp.int32)

    lo = jnp.zeros((B, 1, 1), jnp.int32)
    hi = jnp.full((B, 1, 1), 0x7F800001, jnp.int32)
    lo, hi = probe(jnp.clip(t_hi, lo + 1, hi - 1), lo, hi)
    lo, hi = probe(jnp.clip(t_lo, lo + 1, hi - 1), lo, hi)

    def cond(carry):
        lo, hi = carry
        return jnp.any(hi - lo > 1)

    def body(carry):
        lo, hi = carry
        return probe(lo + (hi - lo) // 2, lo, hi)

    lo, hi = jax.lax.while_loop(cond, body, (lo, hi))
    out_ref[...] = jnp.where(bits >= lo, a, 0.0)


def _reg_kernel(total_elems, rs_ref, cs_ref, out_ref):
    rs = rs_ref[...]
    cs = cs_ref[...]
    s = jnp.sum(rs, axis=2)

    def ent(prob):
        p = prob / s[:, :, None]
        logp = jnp.log(jnp.clip(p, 1e-38, None))
        return -jnp.sum(p * logp, axis=2)

    ex = ent(rs)
    ey = ent(cs)
    tot = jnp.sum(s, axis=1, keepdims=True)
    w = s / tot
    reg = (jnp.sum(s) / total_elems * _LAMBDA_ACT_L1
           + (jnp.mean(ex * w) + jnp.mean(ey * w)) * _LAMBDA_LOCALITY)
    out_ref[...] = reg.reshape(1, 1)


def kernel(x, tau):
    n, c, h, w = x.shape
    nc = n * c
    k_top = max(int(_TOPK * h * w), 1)
    xr = x.reshape(nc, h, w)

    B = 32
    assert nc % B == 0
    grid = (nc // B,)
    sparse, rs, cs = pl.pallas_call(
        functools.partial(_plane_kernel, k_top),
        grid=grid,
        in_specs=[pl.BlockSpec((B, h, w), lambda i: (i, 0, 0))],
        out_specs=[
            pl.BlockSpec((B, h, w), lambda i: (i, 0, 0)),
            pl.BlockSpec((B, h), lambda i: (i, 0)),
            pl.BlockSpec((B, w), lambda i: (i, 0)),
        ],
        out_shape=[
            jax.ShapeDtypeStruct((nc, h, w), x.dtype),
            jax.ShapeDtypeStruct((nc, h), jnp.float32),
            jax.ShapeDtypeStruct((nc, w), jnp.float32),
        ],
        interpret=_INTERPRET,
    )(xr)

    reg2d = pl.pallas_call(
        functools.partial(_reg_kernel, float(nc * h * w)),
        out_shape=jax.ShapeDtypeStruct((1, 1), jnp.float32),
        interpret=_INTERPRET,
    )(rs.reshape(n, c, h), cs.reshape(n, c, w))

    return sparse.reshape(n, c, h, w), reg2d[0, 0]
